# Initial kernel scaffold; baseline (speedup 1.0000x reference)
import functools

import jax
import jax.numpy as jnp
from jax import lax
from jax.experimental import pallas as pl
from jax.experimental.pallas import tpu as pltpu
from jax.experimental.pallas import tpu_sc as plsc

N_NODES = 10000
D_IN = 128
HIDDEN = 128
D_OUT = 64
N_EDGES = 320000

NC = 2
NS = 16
NW = NC * NS
C = 128
NCH = -(-N_EDGES // (NW * C))
E_PAD = NW * NCH * C
N_PAD = 10048
RPT = N_PAD // NS

_MESH = plsc.VectorSubcoreMesh(core_axis_name="c", subcore_axis_name="s")


def _sc_degree(dst_flat, zeros8, ones8):
    @functools.partial(
        pl.kernel,
        out_type=jax.ShapeDtypeStruct((NC * N_PAD, 8), jnp.float32),
        mesh=_MESH,
        scratch_types=[
            pltpu.VMEM((C,), jnp.int32),
            pltpu.VMEM((C, 8), jnp.float32),
            pltpu.VMEM_SHARED((N_PAD, 8), jnp.float32),
        ],
    )
    def kern(dst_hbm, z_hbm, ones_hbm, out_hbm, didx, ones_v, acc):
        c = lax.axis_index("c")
        s = lax.axis_index("s")
        wid = c * NS + s
        r0 = s * RPT
        pltpu.sync_copy(z_hbm.at[pl.ds(r0, RPT)], acc.at[pl.ds(r0, RPT)])
        pltpu.sync_copy(ones_hbm, ones_v)
        plsc.subcore_barrier()

        def body(j, carry):
            base = (wid * NCH + j) * C
            pltpu.sync_copy(dst_hbm.at[pl.ds(base, C)], didx)
            pltpu.sync_copy(ones_v, acc.at[didx], add=True)
            return carry

        lax.fori_loop(0, NCH, body, 0)
        plsc.subcore_barrier()
        pltpu.sync_copy(acc.at[pl.ds(r0, RPT)],
                        out_hbm.at[pl.ds(c * N_PAD + r0, RPT)])

    return kern(dst_flat, zeros8, ones8)


def _sc_aggregate(table, src_flat, dst_flat, zeros_d, d):
    @functools.partial(
        pl.kernel,
        out_type=jax.ShapeDtypeStruct((NC * N_PAD, d), jnp.float32),
        mesh=_MESH,
        scratch_types=[
            pltpu.VMEM((C,), jnp.int32),
            pltpu.VMEM((C,), jnp.int32),
            pltpu.VMEM((C, d), jnp.float32),
            pltpu.VMEM_SHARED((N_PAD, d), jnp.float32),
            pltpu.SemaphoreType.DMA,
        ],
    )
    def kern(tab_hbm, src_hbm, dst_hbm, z_hbm, out_hbm, sidx, didx, rows, acc,
             sem):
        c = lax.axis_index("c")
        s = lax.axis_index("s")
        wid = c * NS + s
        r0 = s * RPT
        pltpu.sync_copy(z_hbm.at[pl.ds(r0, RPT)], acc.at[pl.ds(r0, RPT)])
        plsc.subcore_barrier()

        def body(j, carry):
            base = (wid * NCH + j) * C
            pltpu.sync_copy(src_hbm.at[pl.ds(base, C)], sidx)
            pltpu.sync_copy(dst_hbm.at[pl.ds(base, C)], didx)
            pltpu.async_copy(tab_hbm.at[sidx], rows, sem).wait()
            pltpu.sync_copy(rows, acc.at[didx], add=True)
            return carry

        lax.fori_loop(0, NCH, body, 0)
        plsc.subcore_barrier()
        pltpu.sync_copy(acc.at[pl.ds(r0, RPT)],
                        out_hbm.at[pl.ds(c * N_PAD + r0, RPT)])

    return kern(table, src_flat, dst_flat, zeros_d)


_R = 1000


def _tc_first(x, w1, degp):
    def body(x_ref, w_ref, d_ref, hs_ref, dinv_ref):
        deg = d_ref[0, :, 0:1] + d_ref[1, :, 0:1] + 1.0
        dinv = lax.rsqrt(deg)
        h = jnp.dot(x_ref[...], w_ref[...], preferred_element_type=jnp.float32)
        hs_ref[...] = h * dinv
        dinv_ref[...] = dinv

    return pl.pallas_call(
        body,
        grid=(N_NODES // _R,),
        in_specs=[
            pl.BlockSpec((_R, D_IN), lambda i: (i, 0)),
            pl.BlockSpec((D_IN, HIDDEN), lambda i: (0, 0)),
            pl.BlockSpec((2, _R, 8), lambda i: (0, i, 0)),
        ],
        out_specs=[
            pl.BlockSpec((_R, HIDDEN), lambda i: (i, 0)),
            pl.BlockSpec((_R, 1), lambda i: (i, 0)),
        ],
        out_shape=[
            jax.ShapeDtypeStruct((N_NODES, HIDDEN), jnp.float32),
            jax.ShapeDtypeStruct((N_NODES, 1), jnp.float32),
        ],
    )(x, w1, degp)


def _tc_mid(aggp, hs, dinv, b1, w2):
    def body(a_ref, hs_ref, di_ref, b_ref, w_ref, hs2_ref):
        di = di_ref[...]
        agg = a_ref[0] + a_ref[1]
        h1 = jnp.maximum(di * (agg + hs_ref[...]) + b_ref[...], 0.0)
        hs2_ref[...] = jnp.dot(
            h1, w_ref[...], preferred_element_type=jnp.float32) * di

    return pl.pallas_call(
        body,
        grid=(N_NODES // _R,),
        in_specs=[
            pl.BlockSpec((2, _R, HIDDEN), lambda i: (0, i, 0)),
            pl.BlockSpec((_R, HIDDEN), lambda i: (i, 0)),
            pl.BlockSpec((_R, 1), lambda i: (i, 0)),
            pl.BlockSpec((1, HIDDEN), lambda i: (0, 0)),
            pl.BlockSpec((HIDDEN, D_OUT), lambda i: (0, 0)),
        ],
        out_specs=pl.BlockSpec((_R, D_OUT), lambda i: (i, 0)),
        out_shape=jax.ShapeDtypeStruct((N_NODES, D_OUT), jnp.float32),
    )(aggp, hs, dinv, b1, w2)


def _tc_last(aggp2, hs2, dinv, b2):
    def body(a_ref, hs_ref, di_ref, b_ref, out_ref):
        agg = a_ref[0] + a_ref[1]
        out_ref[...] = di_ref[...] * (agg + hs_ref[...]) + b_ref[...]

    return pl.pallas_call(
        body,
        grid=(N_NODES // _R,),
        in_specs=[
            pl.BlockSpec((2, _R, D_OUT), lambda i: (0, i, 0)),
            pl.BlockSpec((_R, D_OUT), lambda i: (i, 0)),
            pl.BlockSpec((_R, 1), lambda i: (i, 0)),
            pl.BlockSpec((1, D_OUT), lambda i: (0, 0)),
        ],
        out_specs=pl.BlockSpec((_R, D_OUT), lambda i: (i, 0)),
        out_shape=jax.ShapeDtypeStruct((N_NODES, D_OUT), jnp.float32),
    )(aggp2, hs2, dinv, b2)


def kernel(x, edge_index, W1, b1, W2, b2, dropout):
    src = edge_index[0].astype(jnp.int32)
    dst = edge_index[1].astype(jnp.int32)
    pad = E_PAD - N_EDGES
    src_flat = jnp.concatenate([src, jnp.zeros((pad,), jnp.int32)])
    dst_flat = jnp.concatenate([dst, jnp.full((pad,), N_NODES, jnp.int32)])
    zeros8 = jnp.zeros((N_PAD, 8), jnp.float32)
    ones8 = jnp.ones((C, 8), jnp.float32)
    zeros_h = jnp.zeros((N_PAD, HIDDEN), jnp.float32)
    zeros_o = jnp.zeros((N_PAD, D_OUT), jnp.float32)

    degp = _sc_degree(dst_flat, zeros8, ones8).reshape(NC, N_PAD, 8)
    hs, dinv = _tc_first(x, W1, degp[:, :N_NODES])
    aggp = _sc_aggregate(hs, src_flat, dst_flat, zeros_h,
                         HIDDEN).reshape(NC, N_PAD, HIDDEN)
    hs2 = _tc_mid(aggp[:, :N_NODES], hs, dinv, b1.reshape(1, -1), W2)
    aggp2 = _sc_aggregate(hs2, src_flat, dst_flat, zeros_o,
                          D_OUT).reshape(NC, N_PAD, D_OUT)
    return _tc_last(aggp2[:, :N_NODES], hs2, dinv, b2.reshape(1, -1))


# speedup vs baseline: 12.9334x; 12.9334x over previous
import functools

import jax
import jax.numpy as jnp
from jax import lax
from jax.experimental import pallas as pl
from jax.experimental.pallas import tpu as pltpu
from jax.experimental.pallas import tpu_sc as plsc

N_NODES = 10000
D_IN = 128
HIDDEN = 128
D_OUT = 64
N_EDGES = 320000

NC = 2
NS = 16
NW = NC * NS
C = 128
NCH = -(-N_EDGES // (NW * C))
E_PAD = NW * NCH * C
N_PAD = 10112
RPT = N_PAD // NS

_MESH = plsc.VectorSubcoreMesh(
    core_axis_name="c", subcore_axis_name="s", num_cores=NC, num_subcores=NS)


def _sc_degree(dst_flat, zeros8, ones8):
    @functools.partial(
        pl.kernel,
        out_type=jax.ShapeDtypeStruct((NC * N_PAD, 8), jnp.float32),
        mesh=_MESH,
        scratch_types=[
            pltpu.VMEM((C,), jnp.int32),
            pltpu.VMEM((C, 8), jnp.float32),
            pltpu.VMEM_SHARED((N_PAD, 8), jnp.float32),
        ],
        compiler_params=pltpu.CompilerParams(use_tc_tiling_on_sc=False),
    )
    def kern(dst_hbm, z_hbm, ones_hbm, out_hbm, didx, ones_v, acc):
        c = lax.axis_index("c")
        s = lax.axis_index("s")
        wid = c * NS + s
        r0 = s * RPT
        pltpu.sync_copy(z_hbm.at[pl.ds(r0, RPT)], acc.at[pl.ds(r0, RPT)])
        pltpu.sync_copy(ones_hbm, ones_v)
        plsc.subcore_barrier()

        def body(j, carry):
            base = (wid * NCH + j) * C
            pltpu.sync_copy(dst_hbm.at[pl.ds(base, C)], didx)
            pltpu.sync_copy(ones_v, acc.at[didx], add=True)
            return carry

        lax.fori_loop(0, NCH, body, 0)
        plsc.subcore_barrier()
        pltpu.sync_copy(acc.at[pl.ds(r0, RPT)],
                        out_hbm.at[pl.ds(c * N_PAD + r0, RPT)])

    return kern(dst_flat, zeros8, ones8)


def _sc_aggregate(table, src_flat, dst_flat, zeros_d, d):
    @functools.partial(
        pl.kernel,
        out_type=jax.ShapeDtypeStruct((NC * N_PAD, d), jnp.float32),
        mesh=_MESH,
        scratch_types=[
            pltpu.VMEM((C,), jnp.int32),
            pltpu.VMEM((C,), jnp.int32),
            pltpu.VMEM((C, d), jnp.float32),
            pltpu.VMEM_SHARED((N_PAD, d), jnp.float32),
            pltpu.SemaphoreType.DMA,
        ],
        compiler_params=pltpu.CompilerParams(use_tc_tiling_on_sc=False),
    )
    def kern(tab_hbm, src_hbm, dst_hbm, z_hbm, out_hbm, sidx, didx, rows, acc,
             sem):
        c = lax.axis_index("c")
        s = lax.axis_index("s")
        wid = c * NS + s
        r0 = s * RPT
        pltpu.sync_copy(z_hbm.at[pl.ds(r0, RPT)], acc.at[pl.ds(r0, RPT)])
        plsc.subcore_barrier()

        def body(j, carry):
            base = (wid * NCH + j) * C
            pltpu.sync_copy(src_hbm.at[pl.ds(base, C)], sidx)
            pltpu.sync_copy(dst_hbm.at[pl.ds(base, C)], didx)
            pltpu.async_copy(tab_hbm.at[sidx], rows, sem).wait()
            pltpu.sync_copy(rows, acc.at[didx], add=True)
            return carry

        lax.fori_loop(0, NCH, body, 0)
        plsc.subcore_barrier()
        pltpu.sync_copy(acc.at[pl.ds(r0, RPT)],
                        out_hbm.at[pl.ds(c * N_PAD + r0, RPT)])

    return kern(table, src_flat, dst_flat, zeros_d)


_R = 1000


def _tc_first(x, w1, degp):
    def body(x_ref, w_ref, d_ref, hs_ref, dinv_ref):
        deg = d_ref[0, :, 0:1] + d_ref[1, :, 0:1] + 1.0
        dinv = lax.rsqrt(deg)
        h = jnp.dot(x_ref[...], w_ref[...], preferred_element_type=jnp.float32)
        hs_ref[...] = h * dinv
        dinv_ref[...] = dinv

    return pl.pallas_call(
        body,
        grid=(N_NODES // _R,),
        in_specs=[
            pl.BlockSpec((_R, D_IN), lambda i: (i, 0)),
            pl.BlockSpec((D_IN, HIDDEN), lambda i: (0, 0)),
            pl.BlockSpec((2, _R, 8), lambda i: (0, i, 0)),
        ],
        out_specs=[
            pl.BlockSpec((_R, HIDDEN), lambda i: (i, 0)),
            pl.BlockSpec((_R, 1), lambda i: (i, 0)),
        ],
        out_shape=[
            jax.ShapeDtypeStruct((N_NODES, HIDDEN), jnp.float32),
            jax.ShapeDtypeStruct((N_NODES, 1), jnp.float32),
        ],
    )(x, w1, degp)


def _tc_mid(aggp, hs, dinv, b1, w2):
    def body(a_ref, hs_ref, di_ref, b_ref, w_ref, hs2_ref):
        di = di_ref[...]
        agg = a_ref[0] + a_ref[1]
        h1 = jnp.maximum(di * (agg + hs_ref[...]) + b_ref[...], 0.0)
        hs2_ref[...] = jnp.dot(
            h1, w_ref[...], preferred_element_type=jnp.float32) * di

    return pl.pallas_call(
        body,
        grid=(N_NODES // _R,),
        in_specs=[
            pl.BlockSpec((2, _R, HIDDEN), lambda i: (0, i, 0)),
            pl.BlockSpec((_R, HIDDEN), lambda i: (i, 0)),
            pl.BlockSpec((_R, 1), lambda i: (i, 0)),
            pl.BlockSpec((1, HIDDEN), lambda i: (0, 0)),
            pl.BlockSpec((HIDDEN, D_OUT), lambda i: (0, 0)),
        ],
        out_specs=pl.BlockSpec((_R, D_OUT), lambda i: (i, 0)),
        out_shape=jax.ShapeDtypeStruct((N_NODES, D_OUT), jnp.float32),
    )(aggp, hs, dinv, b1, w2)


def _tc_last(aggp2, hs2, dinv, b2):
    def body(a_ref, hs_ref, di_ref, b_ref, out_ref):
        agg = a_ref[0] + a_ref[1]
        out_ref[...] = di_ref[...] * (agg + hs_ref[...]) + b_ref[...]

    return pl.pallas_call(
        body,
        grid=(N_NODES // _R,),
        in_specs=[
            pl.BlockSpec((2, _R, D_OUT), lambda i: (0, i, 0)),
            pl.BlockSpec((_R, D_OUT), lambda i: (i, 0)),
            pl.BlockSpec((_R, 1), lambda i: (i, 0)),
            pl.BlockSpec((1, D_OUT), lambda i: (0, 0)),
        ],
        out_specs=pl.BlockSpec((_R, D_OUT), lambda i: (i, 0)),
        out_shape=jax.ShapeDtypeStruct((N_NODES, D_OUT), jnp.float32),
    )(aggp2, hs2, dinv, b2)


def kernel(x, edge_index, W1, b1, W2, b2, dropout):
    src = edge_index[0].astype(jnp.int32)
    dst = edge_index[1].astype(jnp.int32)
    pad = E_PAD - N_EDGES
    src_flat = jnp.concatenate([src, jnp.zeros((pad,), jnp.int32)])
    dst_flat = jnp.concatenate([dst, jnp.full((pad,), N_NODES, jnp.int32)])
    zeros8 = jnp.zeros((N_PAD, 8), jnp.float32)
    ones8 = jnp.ones((C, 8), jnp.float32)
    zeros_h = jnp.zeros((N_PAD, HIDDEN), jnp.float32)
    zeros_o = jnp.zeros((N_PAD, D_OUT), jnp.float32)

    degp = _sc_degree(dst_flat, zeros8, ones8).reshape(NC, N_PAD, 8)
    hs, dinv = _tc_first(x, W1, degp[:, :N_NODES])
    aggp = _sc_aggregate(hs, src_flat, dst_flat, zeros_h,
                         HIDDEN).reshape(NC, N_PAD, HIDDEN)
    hs2 = _tc_mid(aggp[:, :N_NODES], hs, dinv, b1.reshape(1, -1), W2)
    aggp2 = _sc_aggregate(hs2, src_flat, dst_flat, zeros_o,
                          D_OUT).reshape(NC, N_PAD, D_OUT)
    return _tc_last(aggp2[:, :N_NODES], hs2, dinv, b2.reshape(1, -1))
